# baseline (speedup 1.0000x reference)
import functools

import jax
import jax.numpy as jnp
from jax import lax
from jax.experimental import pallas as pl
from jax.experimental.pallas import tpu as pltpu
from jax.experimental.pallas import tpu_sc as plsc

_B = 2
_C = 22
_NCLS = _C - 1
_HS, _WS = 120, 160
_P = _HS * _WS
_NSTEP = 8
_NBINS = _P
_ACC_N = _B * _NCLS * _NBINS
_NV = _B * _NSTEP * _P
_NC, _NS = 2, 16
_NW = _NC * _NS
_NVB = _NSTEP * _P
_CHUNK = 120
_NCHUNK = _NVB // (_NW * _CHUNK)
_ACC_NB = _NCLS * _NBINS
_ACC_PAD = 409600
_SLICE = _ACC_PAD // _NS
_ZB = 6400


_HB = 32
_RPS = _HB // 4
_NG = 480 // _HB


def _prep_body(lab_ref, vp_ref, vidx_ref, vval_ref, stats_ref):
    b = pl.program_id(0)
    g = pl.program_id(1)
    ws_acc = jnp.zeros((_NCLS, 1), jnp.float32)
    wz_acc = jnp.zeros((_NCLS, 1), jnp.float32)
    s_i = lax.broadcasted_iota(jnp.int32, (640, _WS), 0)
    s_j = lax.broadcasted_iota(jnp.int32, (640, _WS), 1)
    sel4 = (s_i == 4 * s_j).astype(jnp.float32)
    x_all = jnp.concatenate(
        [lab_ref[0, :, 4 * r, :] for r in range(_RPS)]
        + [vp_ref[0, :, 4 * r, :] for r in range(_RPS)], axis=0)
    xd = lax.dot(x_all, sel4, precision=lax.Precision.HIGHEST,
                 preferred_element_type=jnp.float32)
    row_iota = lax.broadcasted_iota(jnp.int32, (_C, _WS), 0)
    row_iota66 = lax.broadcasted_iota(jnp.int32, (3 * _C, _WS), 0)
    t8 = ((lax.broadcasted_iota(jnp.int32, (_NSTEP, 1), 0) + 1) * 8).astype(jnp.float32)
    x_iota = lax.broadcasted_iota(jnp.int32, (1, _WS), 1)
    xs = (x_iota * 4).astype(jnp.float32)
    cls_iota = lax.broadcasted_iota(jnp.int32, (_NCLS, _WS), 0) + 1
    for r in range(_RPS):
        lab = xd[_C * r:_C * r + _C]
        m = jnp.max(lab, axis=0, keepdims=True)
        esum = jnp.sum(jnp.exp(lab - m), axis=0, keepdims=True)
        prob = 1.0 / esum
        labels = jnp.min(jnp.where(lab == m, row_iota, _C), axis=0, keepdims=True)

        vp = xd[_C * _RPS + 3 * _C * r:_C * _RPS + 3 * _C * (r + 1)]
        lab3 = 3 * labels
        nx = jnp.sum(jnp.where(row_iota66 == lab3, vp, 0.0), axis=0, keepdims=True)
        ny = jnp.sum(jnp.where(row_iota66 == lab3 + 1, vp, 0.0), axis=0, keepdims=True)
        vz = jnp.sum(jnp.where(row_iota66 == lab3 + 2, vp, 0.0), axis=0, keepdims=True)

        nrm = jnp.sqrt(nx * nx + ny * ny) + 1e-6
        nxn = nx / nrm
        nyn = ny / nrm
        z = jnp.exp(jnp.clip(vz, -3.0, 3.0))
        w = jnp.where(labels >= 1, prob, 0.0)

        ys = jnp.zeros((1, _WS), jnp.float32) + (_HB * g + 4 * r).astype(jnp.float32)
        cm = jnp.maximum(labels - 1, 0)
        base = (b * _NCLS + cm) * _NBINS
        ix = jnp.clip(((xs + t8 * nxn) * 0.25).astype(jnp.int32), 0, _WS - 1)
        iy = jnp.clip(((ys + t8 * nyn) * 0.25).astype(jnp.int32), 0, _HS - 1)
        vidx_ref[0, r, :, :] = base + iy * _WS + ix
        vval_ref[0, r, :, :] = jnp.broadcast_to(w, (_NSTEP, _WS))

        selm = jnp.broadcast_to(labels, (_NCLS, _WS)) == cls_iota
        wm = jnp.where(selm, jnp.broadcast_to(prob, (_NCLS, _WS)), 0.0)
        wzm = jnp.where(selm, jnp.broadcast_to(prob * z, (_NCLS, _WS)), 0.0)
        ws_acc = ws_acc + jnp.sum(wm, axis=1, keepdims=True)
        wz_acc = wz_acc + jnp.sum(wzm, axis=1, keepdims=True)

    @pl.when(g == 0)
    def _init():
        stats_ref[0, :, 0:1] = ws_acc
        stats_ref[0, :, 1:2] = wz_acc

    @pl.when(g != 0)
    def _acc():
        stats_ref[0, :, 0:1] += ws_acc
        stats_ref[0, :, 1:2] += wz_acc


def _prep(label_b, vertex_b):
    nb = label_b.shape[0]
    return pl.pallas_call(
        _prep_body,
        grid=(nb, _NG),
        in_specs=[
            pl.BlockSpec((1, _C, _HB, 640), lambda b, g: (b, 0, g, 0)),
            pl.BlockSpec((1, 3 * _C, _HB, 640), lambda b, g: (b, 0, g, 0)),
        ],
        out_specs=[
            pl.BlockSpec((1, _RPS, _NSTEP, _WS), lambda b, g: (b, g, 0, 0)),
            pl.BlockSpec((1, _RPS, _NSTEP, _WS), lambda b, g: (b, g, 0, 0)),
            pl.BlockSpec((1, _NCLS, 2), lambda b, g: (b, 0, 0)),
        ],
        out_shape=[
            jax.ShapeDtypeStruct((nb, _HS, _NSTEP, _WS), jnp.int32),
            jax.ShapeDtypeStruct((nb, _HS, _NSTEP, _WS), jnp.float32),
            jax.ShapeDtypeStruct((nb, _NCLS, 2), jnp.float32),
        ],
    )(label_b, vertex_b)


def _scatter_body(vidx_hbm, vval_hbm, out_hbm, idx_v, val_v, zbuf, acc_sh):
    cid = lax.axis_index("c")
    sid = lax.axis_index("s")
    wid = sid * _NC + cid

    def _zb(i, carry):
        zbuf[pl.ds(i * 16, 16)] = jnp.zeros((16,), jnp.float32)
        return carry

    lax.fori_loop(0, _ZB // 16, _zb, 0)
    for k in range(_SLICE // _ZB):
        pltpu.sync_copy(zbuf, acc_sh.at[pl.ds(sid * _SLICE + k * _ZB, _ZB)])
    plsc.subcore_barrier()

    pltpu.sync_copy(vidx_hbm.at[wid], idx_v)
    pltpu.sync_copy(vval_hbm.at[wid], val_v)
    for j in range(_NCHUNK):
        pltpu.sync_copy(val_v.at[j], acc_sh.at[idx_v.at[j]], add=True)
    plsc.subcore_barrier()
    pltpu.sync_copy(acc_sh.at[pl.ds(sid * _SLICE, _SLICE)],
                    out_hbm.at[cid, 0, pl.ds(sid * _SLICE, _SLICE)])


def _make_scatter():
    return functools.partial(
        pl.kernel,
        out_type=jax.ShapeDtypeStruct((_NC, 1, _ACC_PAD), jnp.float32),
        mesh=plsc.VectorSubcoreMesh(core_axis_name="c", subcore_axis_name="s",
                                    num_cores=_NC, num_subcores=_NS),
        scratch_types=[
            pltpu.VMEM((_NCHUNK, _CHUNK), jnp.int32),
            pltpu.VMEM((_NCHUNK, _CHUNK), jnp.float32),
            pltpu.VMEM((_ZB,), jnp.float32),
            pltpu.VMEM_SHARED((_ACC_PAD,), jnp.float32),
        ],
    )(_scatter_body)


def _fin_body(acc0_ref, acc1_ref, stats_ref, ext_ref, md_ref, roi_ref, pose_ref):
    rows = []
    for acc_ref in (acc0_ref, acc1_ref):
        for c in range(_NCLS):
            off = c * _NBINS
            rows.append(acc_ref[0, 0:1, off:off + _NBINS]
                        + acc_ref[1, 0:1, off:off + _NBINS])
    a = jnp.concatenate(rows, axis=0)
    nr = _B * _NCLS
    mx = jnp.max(a, axis=1, keepdims=True)
    lin = lax.broadcasted_iota(jnp.int32, (nr, _NBINS), 1)
    am = jnp.min(jnp.where(a == mx, lin, _NBINS), axis=1, keepdims=True)
    cy = (am // _WS).astype(jnp.float32) * 4.0
    cx = (am % _WS).astype(jnp.float32) * 4.0

    ws = jnp.concatenate([stats_ref[0, :, 0:1], stats_ref[1, :, 0:1]], axis=0) + 1e-6
    wz = jnp.concatenate([stats_ref[0, :, 1:2], stats_ref[1, :, 1:2]], axis=0)
    zbar = wz / ws
    emax = jnp.max(ext_ref[...], axis=1, keepdims=True)[1:_C]
    em2 = jnp.concatenate([emax, emax], axis=0)
    riota = lax.broadcasted_iota(jnp.int32, (nr, 1), 0)
    isb1 = riota >= _NCLS
    fx = jnp.where(isb1, md_ref[1, 0], md_ref[0, 0])
    fy = jnp.where(isb1, md_ref[1, 4], md_ref[0, 4])
    pxc = jnp.where(isb1, md_ref[1, 2], md_ref[0, 2])
    pyc = jnp.where(isb1, md_ref[1, 5], md_ref[0, 5])
    bw = fx * em2 / zbar
    bh = fy * em2 / zbar
    valid = (mx > 100.0).astype(jnp.float32)
    cls = (riota % _NCLS + 1).astype(jnp.float32)

    roi_cols = [cls, cx - bw * 0.5, cy - bh * 0.5, cx + bw * 0.5,
                cy + bh * 0.5, mx, valid]
    pose_cols = [(cx - pxc) * zbar / fx, (cy - pyc) * zbar / fy,
                 zbar, mx / (ws + float(_NSTEP))]
    for k, col in enumerate(roi_cols):
        roi_ref[0, :, k:k + 1] = col[0:_NCLS]
        roi_ref[1, :, k:k + 1] = col[_NCLS:nr]
    for k, col in enumerate(pose_cols):
        pose_ref[0, :, k:k + 1] = col[0:_NCLS]
        pose_ref[1, :, k:k + 1] = col[_NCLS:nr]


def _finalize(acc0, acc1, stats, extents, mdata):
    return pl.pallas_call(
        _fin_body,
        out_shape=[
            jax.ShapeDtypeStruct((_B, _NCLS, 7), jnp.float32),
            jax.ShapeDtypeStruct((_B, _NCLS, 4), jnp.float32),
        ],
    )(acc0, acc1, stats, extents, mdata)


def kernel(label_2d, vertex_pred, extents, poses, mdata):
    accs, stats_list = [], []
    scatter = _make_scatter()
    for bi in range(_B):
        vidx, vval, st = _prep(label_2d[bi:bi + 1], vertex_pred[bi:bi + 1])
        acc_b = scatter(vidx.reshape(_NW, _NCHUNK, _CHUNK),
                        vval.reshape(_NW, _NCHUNK, _CHUNK))
        accs.append(acc_b)
        stats_list.append(st)
    stats = jnp.concatenate(stats_list, axis=0)
    return _finalize(accs[0], accs[1], stats, extents, mdata)


# speedup vs baseline: 1.9081x; 1.9081x over previous
import functools

import jax
import jax.numpy as jnp
from jax import lax
from jax.experimental import pallas as pl
from jax.experimental.pallas import tpu as pltpu
from jax.experimental.pallas import tpu_sc as plsc

_B = 2
_C = 22
_NCLS = _C - 1
_HS, _WS = 120, 160
_P = _HS * _WS
_NSTEP = 8
_NBINS = _P
_ACC_N = _B * _NCLS * _NBINS
_NV = _B * _NSTEP * _P
_NC, _NS = 2, 16
_NW = _NC * _NS
_CHUNK = 128
_NCHUNK = _NV // (_NW * _CHUNK)
_ACC_PAD = 819200
_SLICE = _ACC_PAD // _NS
_ZB = 6400


_HB = 32
_RPS = _HB // 4
_NG = 480 // _HB


def _prep_body(lab_ref, vp_ref, vidx_ref, vval_ref, stats_ref):
    b = pl.program_id(0)
    g = pl.program_id(1)
    ws_acc = jnp.zeros((_NCLS, 1), jnp.float32)
    wz_acc = jnp.zeros((_NCLS, 1), jnp.float32)
    s_i = lax.broadcasted_iota(jnp.int32, (640, _WS), 0)
    s_j = lax.broadcasted_iota(jnp.int32, (640, _WS), 1)
    sel4 = (s_i == 4 * s_j).astype(jnp.float32)
    x_all = jnp.concatenate(
        [lab_ref[0, :, 4 * r, :] for r in range(_RPS)]
        + [vp_ref[0, :, 4 * r, :] for r in range(_RPS)], axis=0)
    xd = lax.dot(x_all, sel4, precision=lax.Precision.HIGHEST,
                 preferred_element_type=jnp.float32)
    row_iota = lax.broadcasted_iota(jnp.int32, (_C, _WS), 0)
    row_iota66 = lax.broadcasted_iota(jnp.int32, (3 * _C, _WS), 0)
    t8 = ((lax.broadcasted_iota(jnp.int32, (_NSTEP, 1), 0) + 1) * 8).astype(jnp.float32)
    x_iota = lax.broadcasted_iota(jnp.int32, (1, _WS), 1)
    xs = (x_iota * 4).astype(jnp.float32)
    cls_iota = lax.broadcasted_iota(jnp.int32, (_NCLS, _WS), 0) + 1
    for r in range(_RPS):
        lab = xd[_C * r:_C * r + _C]
        m = jnp.max(lab, axis=0, keepdims=True)
        esum = jnp.sum(jnp.exp(lab - m), axis=0, keepdims=True)
        prob = 1.0 / esum
        labels = jnp.min(jnp.where(lab == m, row_iota, _C), axis=0, keepdims=True)

        vp = xd[_C * _RPS + 3 * _C * r:_C * _RPS + 3 * _C * (r + 1)]
        lab3 = 3 * labels
        nx = jnp.sum(jnp.where(row_iota66 == lab3, vp, 0.0), axis=0, keepdims=True)
        ny = jnp.sum(jnp.where(row_iota66 == lab3 + 1, vp, 0.0), axis=0, keepdims=True)
        vz = jnp.sum(jnp.where(row_iota66 == lab3 + 2, vp, 0.0), axis=0, keepdims=True)

        nrm = jnp.sqrt(nx * nx + ny * ny) + 1e-6
        nxn = nx / nrm
        nyn = ny / nrm
        z = jnp.exp(jnp.clip(vz, -3.0, 3.0))
        w = jnp.where(labels >= 1, prob, 0.0)

        ys = jnp.zeros((1, _WS), jnp.float32) + (_HB * g + 4 * r).astype(jnp.float32)
        cm = jnp.maximum(labels - 1, 0)
        base = (b * _NCLS + cm) * _NBINS
        ix = jnp.clip(((xs + t8 * nxn) * 0.25).astype(jnp.int32), 0, _WS - 1)
        iy = jnp.clip(((ys + t8 * nyn) * 0.25).astype(jnp.int32), 0, _HS - 1)
        vidx_ref[0, r, :, :] = base + iy * _WS + ix
        vval_ref[0, r, :, :] = jnp.broadcast_to(w, (_NSTEP, _WS))

        selm = jnp.broadcast_to(labels, (_NCLS, _WS)) == cls_iota
        wm = jnp.where(selm, jnp.broadcast_to(prob, (_NCLS, _WS)), 0.0)
        wzm = jnp.where(selm, jnp.broadcast_to(prob * z, (_NCLS, _WS)), 0.0)
        ws_acc = ws_acc + jnp.sum(wm, axis=1, keepdims=True)
        wz_acc = wz_acc + jnp.sum(wzm, axis=1, keepdims=True)

    @pl.when(g == 0)
    def _init():
        stats_ref[0, :, 0:1] = ws_acc
        stats_ref[0, :, 1:2] = wz_acc

    @pl.when(g != 0)
    def _acc():
        stats_ref[0, :, 0:1] += ws_acc
        stats_ref[0, :, 1:2] += wz_acc


def _prep(label_2d, vertex_pred):
    return pl.pallas_call(
        _prep_body,
        grid=(_B, _NG),
        in_specs=[
            pl.BlockSpec((1, _C, _HB, 640), lambda b, g: (b, 0, g, 0)),
            pl.BlockSpec((1, 3 * _C, _HB, 640), lambda b, g: (b, 0, g, 0)),
        ],
        out_specs=[
            pl.BlockSpec((1, _RPS, _NSTEP, _WS), lambda b, g: (b, g, 0, 0)),
            pl.BlockSpec((1, _RPS, _NSTEP, _WS), lambda b, g: (b, g, 0, 0)),
            pl.BlockSpec((1, _NCLS, 2), lambda b, g: (b, 0, 0)),
        ],
        out_shape=[
            jax.ShapeDtypeStruct((_B, _HS, _NSTEP, _WS), jnp.int32),
            jax.ShapeDtypeStruct((_B, _HS, _NSTEP, _WS), jnp.float32),
            jax.ShapeDtypeStruct((_B, _NCLS, 2), jnp.float32),
        ],
    )(label_2d, vertex_pred)


def _scatter_body(vidx_hbm, vval_hbm, out_hbm, idx_v, val_v, zbuf, acc_sh):
    cid = lax.axis_index("c")
    sid = lax.axis_index("s")
    wid = sid * _NC + cid

    def _zb(i, carry):
        zbuf[pl.ds(i * 16, 16)] = jnp.zeros((16,), jnp.float32)
        return carry

    lax.fori_loop(0, _ZB // 16, _zb, 0)
    for k in range(_SLICE // _ZB):
        pltpu.sync_copy(zbuf, acc_sh.at[pl.ds(sid * _SLICE + k * _ZB, _ZB)])
    plsc.subcore_barrier()

    pltpu.sync_copy(vidx_hbm.at[wid], idx_v)
    pltpu.sync_copy(vval_hbm.at[wid], val_v)
    for j in range(_NCHUNK):
        pltpu.sync_copy(val_v.at[j], acc_sh.at[idx_v.at[j]], add=True)
    plsc.subcore_barrier()
    pltpu.sync_copy(acc_sh.at[pl.ds(sid * _SLICE, _SLICE)],
                    out_hbm.at[cid, 0, pl.ds(sid * _SLICE, _SLICE)])


def _make_scatter():
    return functools.partial(
        pl.kernel,
        out_type=jax.ShapeDtypeStruct((_NC, 1, _ACC_PAD), jnp.float32),
        mesh=plsc.VectorSubcoreMesh(core_axis_name="c", subcore_axis_name="s",
                                    num_cores=_NC, num_subcores=_NS),
        scratch_types=[
            pltpu.VMEM((_NCHUNK, _CHUNK), jnp.int32),
            pltpu.VMEM((_NCHUNK, _CHUNK), jnp.float32),
            pltpu.VMEM((_ZB,), jnp.float32),
            pltpu.VMEM_SHARED((_ACC_PAD,), jnp.float32),
        ],
    )(_scatter_body)


def _fin_body(acc_ref, stats_ref, ext_ref, md_ref, roi_ref, pose_ref):
    rows = []
    for b in range(_B):
        for c in range(_NCLS):
            off = (b * _NCLS + c) * _NBINS
            rows.append(acc_ref[0, 0:1, off:off + _NBINS]
                        + acc_ref[1, 0:1, off:off + _NBINS])
    a = jnp.concatenate(rows, axis=0)
    nr = _B * _NCLS
    mx = jnp.max(a, axis=1, keepdims=True)
    lin = lax.broadcasted_iota(jnp.int32, (nr, _NBINS), 1)
    am = jnp.min(jnp.where(a == mx, lin, _NBINS), axis=1, keepdims=True)
    cy = (am // _WS).astype(jnp.float32) * 4.0
    cx = (am % _WS).astype(jnp.float32) * 4.0

    ws = jnp.concatenate([stats_ref[0, :, 0:1], stats_ref[1, :, 0:1]], axis=0) + 1e-6
    wz = jnp.concatenate([stats_ref[0, :, 1:2], stats_ref[1, :, 1:2]], axis=0)
    zbar = wz / ws
    emax = jnp.max(ext_ref[...], axis=1, keepdims=True)[1:_C]
    em2 = jnp.concatenate([emax, emax], axis=0)
    riota = lax.broadcasted_iota(jnp.int32, (nr, 1), 0)
    isb1 = riota >= _NCLS
    fx = jnp.where(isb1, md_ref[1, 0], md_ref[0, 0])
    fy = jnp.where(isb1, md_ref[1, 4], md_ref[0, 4])
    pxc = jnp.where(isb1, md_ref[1, 2], md_ref[0, 2])
    pyc = jnp.where(isb1, md_ref[1, 5], md_ref[0, 5])
    bw = fx * em2 / zbar
    bh = fy * em2 / zbar
    valid = (mx > 100.0).astype(jnp.float32)
    cls = (riota % _NCLS + 1).astype(jnp.float32)

    roi_cols = [cls, cx - bw * 0.5, cy - bh * 0.5, cx + bw * 0.5,
                cy + bh * 0.5, mx, valid]
    pose_cols = [(cx - pxc) * zbar / fx, (cy - pyc) * zbar / fy,
                 zbar, mx / (ws + float(_NSTEP))]
    for k, col in enumerate(roi_cols):
        roi_ref[0, :, k:k + 1] = col[0:_NCLS]
        roi_ref[1, :, k:k + 1] = col[_NCLS:nr]
    for k, col in enumerate(pose_cols):
        pose_ref[0, :, k:k + 1] = col[0:_NCLS]
        pose_ref[1, :, k:k + 1] = col[_NCLS:nr]


def _finalize(acc2, stats, extents, mdata):
    return pl.pallas_call(
        _fin_body,
        out_shape=[
            jax.ShapeDtypeStruct((_B, _NCLS, 7), jnp.float32),
            jax.ShapeDtypeStruct((_B, _NCLS, 4), jnp.float32),
        ],
    )(acc2, stats, extents, mdata)


def kernel(label_2d, vertex_pred, extents, poses, mdata):
    vidx, vval, stats = _prep(label_2d, vertex_pred)
    acc2 = _make_scatter()(vidx.reshape(_NW, _NCHUNK, _CHUNK),
                           vval.reshape(_NW, _NCHUNK, _CHUNK))
    return _finalize(acc2, stats, extents, mdata)
